# baseline (speedup 1.0000x reference)
import functools

import jax
import jax.numpy as jnp
from jax import lax
from jax.experimental import pallas as pl
from jax.experimental.pallas import tpu as pltpu
from jax.experimental.pallas import tpu_sc as plsc

D = 64
L = 50
B = 4096
V = 100000
NC = 2
NS = 16
NW = NC * NS
PASSES = D // NW
VR = B // 16


def _build_kernel():
    mesh = plsc.VectorSubcoreMesh(core_axis_name="c", subcore_axis_name="s")

    @functools.partial(
        pl.kernel,
        mesh=mesh,
        compiler_params=pltpu.CompilerParams(
            use_tc_tiling_on_sc=False, needs_layout_passes=False),
        out_type=jax.ShapeDtypeStruct((L, D, B), jnp.float32),
        scratch_types=[
            pltpu.VMEM((V,), jnp.float32),
            pltpu.VMEM((B,), jnp.int32),
            pltpu.VMEM((B,), jnp.int32),
            pltpu.VMEM((B,), jnp.float32),
            pltpu.VMEM((B,), jnp.float32),
            pltpu.VMEM((L, 16), jnp.float32),
            pltpu.SemaphoreType.DMA,
            pltpu.SemaphoreType.DMA,
            pltpu.SemaphoreType.DMA,
            pltpu.SemaphoreType.DMA,
        ],
    )
    def gather_add(ids_hbm, feat_hbm, posb_hbm, out_hbm,
                   trow, ib0, ib1, st0, st1, pbv, is0, is1, ws0, ws1):
        ibufs, stgs = (ib0, ib1), (st0, st1)
        isems, wsems = (is0, is1), (ws0, ws1)
        wid = lax.axis_index("s") * NC + lax.axis_index("c")

        def pass_body(f, carry):
            d = wid + f * NW
            pltpu.sync_copy(feat_hbm.at[d], trow)
            pltpu.sync_copy(posb_hbm.at[d], pbv)
            pltpu.async_copy(ids_hbm.at[0], ibufs[0], isems[0])

            def row_pair(go, carry2):
                for b in range(2):
                    l = go * 2 + b
                    @pl.when(l >= 2)
                    def _():
                        pltpu.make_async_copy(
                            stgs[b], out_hbm.at[l - 2, d], wsems[b]).wait()

                    pltpu.make_async_copy(
                        ids_hbm.at[0], ibufs[b], isems[b]).wait()

                    @pl.when(l + 1 < L)
                    def _():
                        pltpu.async_copy(
                            ids_hbm.at[l + 1], ibufs[1 - b], isems[1 - b])

                    pv = pbv[l, pl.ds(0, 16)]
                    for k in range(VR):
                        sl = pl.ds(k * 16, 16)
                        iv = ibufs[b][sl]
                        stgs[b][sl] = plsc.load_gather(trow, [iv]) + pv
                    pltpu.async_copy(stgs[b], out_hbm.at[l, d], wsems[b])
                return carry2

            lax.fori_loop(0, L // 2, row_pair, 0)
            pltpu.make_async_copy(stgs[0], out_hbm.at[L - 2, d], wsems[0]).wait()
            pltpu.make_async_copy(stgs[1], out_hbm.at[L - 1, d], wsems[1]).wait()
            return carry

        lax.fori_loop(0, PASSES, pass_body, 0)

    return gather_add


def kernel(phrase, phrase_emb_weight, pos_emb_weight):
    Bsz, Lseq = phrase.shape
    ids = phrase.T.astype(jnp.int32)
    feat = phrase_emb_weight.T
    pos = pos_emb_weight[:Lseq]
    posb = jnp.broadcast_to(pos.T[:, :, None], (pos.shape[1], Lseq, 16))
    out_phys = _build_kernel()(ids, feat, posb)
    return jnp.transpose(out_phys, (2, 0, 1))


# speedup vs baseline: 1.4336x; 1.4336x over previous
import functools

import jax
import jax.numpy as jnp
from jax import lax
from jax.experimental import pallas as pl
from jax.experimental.pallas import tpu as pltpu
from jax.experimental.pallas import tpu_sc as plsc

D = 64
L = 50
B = 4096
V = 100000
NC = 2
NS = 16
NW = NC * NS
PASSES = D // NW
VR = B // 16


def _build_kernel():
    mesh = plsc.VectorSubcoreMesh(core_axis_name="c", subcore_axis_name="s")

    @functools.partial(
        pl.kernel,
        mesh=mesh,
        compiler_params=pltpu.CompilerParams(needs_layout_passes=False),
        out_type=jax.ShapeDtypeStruct((L, D, B), jnp.float32),
        scratch_types=[
            pltpu.VMEM((V,), jnp.float32),
            pltpu.VMEM((B,), jnp.int32),
            pltpu.VMEM((B,), jnp.int32),
            pltpu.VMEM((B,), jnp.float32),
            pltpu.VMEM((B,), jnp.float32),
            pltpu.VMEM((L, 16), jnp.float32),
            pltpu.SemaphoreType.DMA,
            pltpu.SemaphoreType.DMA,
            pltpu.SemaphoreType.DMA,
            pltpu.SemaphoreType.DMA,
        ],
    )
    def gather_add(ids_hbm, feat_hbm, posb_hbm, out_hbm,
                   trow, ib0, ib1, st0, st1, pbv, is0, is1, ws0, ws1):
        ibufs, stgs = (ib0, ib1), (st0, st1)
        isems, wsems = (is0, is1), (ws0, ws1)
        wid = lax.axis_index("s") * NC + lax.axis_index("c")

        def pass_body(f, carry):
            d = wid + f * NW
            pltpu.sync_copy(feat_hbm.at[d], trow)
            pltpu.sync_copy(posb_hbm.at[d], pbv)
            pltpu.async_copy(ids_hbm.at[0], ibufs[0], isems[0])

            def row_pair(go, carry2):
                for b in range(2):
                    l = go * 2 + b
                    @pl.when(l >= 2)
                    def _():
                        pltpu.make_async_copy(
                            stgs[b], out_hbm.at[l - 2, d], wsems[b]).wait()

                    pltpu.make_async_copy(
                        ids_hbm.at[0], ibufs[b], isems[b]).wait()

                    @pl.when(l + 1 < L)
                    def _():
                        pltpu.async_copy(
                            ids_hbm.at[l + 1], ibufs[1 - b], isems[1 - b])

                    pv = pbv[l, pl.ds(0, 16)]
                    for k in range(VR):
                        sl = pl.ds(k * 16, 16)
                        iv = ibufs[b][sl]
                        stgs[b][sl] = plsc.load_gather(trow, [iv]) + pv
                    pltpu.async_copy(stgs[b], out_hbm.at[l, d], wsems[b])
                return carry2

            lax.fori_loop(0, L // 2, row_pair, 0)
            pltpu.make_async_copy(stgs[0], out_hbm.at[L - 2, d], wsems[0]).wait()
            pltpu.make_async_copy(stgs[1], out_hbm.at[L - 1, d], wsems[1]).wait()
            return carry

        lax.fori_loop(0, PASSES, pass_body, 0)

    return gather_add


def kernel(phrase, phrase_emb_weight, pos_emb_weight):
    Bsz, Lseq = phrase.shape
    ids = phrase.T.astype(jnp.int32)
    feat = phrase_emb_weight.T
    pos = pos_emb_weight[:Lseq]
    posb = jnp.broadcast_to(pos.T[:, :, None], (pos.shape[1], Lseq, 16))
    out_phys = _build_kernel()(ids, feat, posb)
    return jnp.transpose(out_phys, (2, 0, 1))


# speedup vs baseline: 2.1305x; 1.4861x over previous
import functools

import jax
import jax.numpy as jnp
from jax import lax
from jax.experimental import pallas as pl
from jax.experimental.pallas import tpu as pltpu
from jax.experimental.pallas import tpu_sc as plsc

D = 64
L = 50
B = 4096
V = 100000
NC = 2
NS = 16
NW = NC * NS
PASSES = D // NW
VR = B // 16


def _build_kernel():
    mesh = plsc.VectorSubcoreMesh(core_axis_name="c", subcore_axis_name="s")

    @functools.partial(
        pl.kernel,
        mesh=mesh,
        compiler_params=pltpu.CompilerParams(needs_layout_passes=False),
        out_type=jax.ShapeDtypeStruct((L, D, B), jnp.float32),
        scratch_types=[
            pltpu.VMEM((V,), jnp.float32),
            pltpu.VMEM((B,), jnp.int32),
            pltpu.VMEM((B,), jnp.int32),
            pltpu.VMEM((B,), jnp.float32),
            pltpu.VMEM((B,), jnp.float32),
            pltpu.VMEM((L, 16), jnp.float32),
            pltpu.SemaphoreType.DMA,
            pltpu.SemaphoreType.DMA,
            pltpu.SemaphoreType.DMA,
            pltpu.SemaphoreType.DMA,
        ],
    )
    def gather_add(ids_hbm, feat_hbm, posb_hbm, out_hbm,
                   trow, ib0, ib1, st0, st1, pbv, is0, is1, ws0, ws1):
        ibufs, stgs = (ib0, ib1), (st0, st1)
        isems, wsems = (is0, is1), (ws0, ws1)
        wid = lax.axis_index("s") * NC + lax.axis_index("c")

        def pass_body(f, carry):
            d = wid + f * NW
            pltpu.sync_copy(feat_hbm.at[d], trow)
            pltpu.sync_copy(posb_hbm.at[d], pbv)
            pltpu.async_copy(ids_hbm.at[0], ibufs[0], isems[0])

            def row_pair(go, carry2):
                for b in range(2):
                    l = go * 2 + b
                    @pl.when(l >= 2)
                    def _():
                        pltpu.make_async_copy(
                            stgs[b], out_hbm.at[l - 2, d], wsems[b]).wait()

                    pltpu.make_async_copy(
                        ids_hbm.at[0], ibufs[b], isems[b]).wait()

                    @pl.when(l + 1 < L)
                    def _():
                        pltpu.async_copy(
                            ids_hbm.at[l + 1], ibufs[1 - b], isems[1 - b])

                    pv = pbv[l, pl.ds(0, 16)]

                    @plsc.parallel_loop(0, VR, 1, unroll=8)
                    def _(k):
                        sl = pl.ds(k * 16, 16)
                        iv = ibufs[b][sl]
                        stgs[b][sl] = plsc.load_gather(trow, [iv]) + pv
                    pltpu.async_copy(stgs[b], out_hbm.at[l, d], wsems[b])
                return carry2

            lax.fori_loop(0, L // 2, row_pair, 0)
            pltpu.make_async_copy(stgs[0], out_hbm.at[L - 2, d], wsems[0]).wait()
            pltpu.make_async_copy(stgs[1], out_hbm.at[L - 1, d], wsems[1]).wait()
            return carry

        lax.fori_loop(0, PASSES, pass_body, 0)

    return gather_add


def kernel(phrase, phrase_emb_weight, pos_emb_weight):
    Bsz, Lseq = phrase.shape
    ids = phrase.T.astype(jnp.int32)
    feat = phrase_emb_weight.T
    pos = pos_emb_weight[:Lseq]
    posb = jnp.broadcast_to(pos.T[:, :, None], (pos.shape[1], Lseq, 16))
    out_phys = _build_kernel()(ids, feat, posb)
    return jnp.transpose(out_phys, (2, 0, 1))


# speedup vs baseline: 2.8431x; 1.3345x over previous
import functools

import jax
import jax.numpy as jnp
from jax import lax
from jax.experimental import pallas as pl
from jax.experimental.pallas import tpu as pltpu
from jax.experimental.pallas import tpu_sc as plsc

D = 64
L = 50
B = 4096
V = 100000
NC = 2
NS = 16
NW = NC * NS
PASSES = D // NW
VR = B // 16
NIB = 3
NST = 2
STEP = 6


def _build_kernel():
    mesh = plsc.VectorSubcoreMesh(core_axis_name="c", subcore_axis_name="s")

    @functools.partial(
        pl.kernel,
        mesh=mesh,
        compiler_params=pltpu.CompilerParams(needs_layout_passes=False),
        out_type=jax.ShapeDtypeStruct((L, D, B), jnp.float32),
        scratch_types=(
            [pltpu.VMEM((V,), jnp.float32)]
            + [pltpu.VMEM((B,), jnp.int32) for _ in range(NIB)]
            + [pltpu.VMEM((B,), jnp.float32) for _ in range(NST)]
            + [pltpu.VMEM((L, 16), jnp.float32)]
            + [pltpu.SemaphoreType.DMA for _ in range(NIB + NST)]
        ),
    )
    def gather_add(ids_hbm, feat_hbm, posb_hbm, out_hbm, trow, *rest):
        ibufs = rest[:NIB]
        stgs = rest[NIB:NIB + NST]
        pbv = rest[NIB + NST]
        isems = rest[NIB + NST + 1:NIB + NST + 1 + NIB]
        wsems = rest[NIB + NST + 1 + NIB:]
        wid = lax.axis_index("s") * NC + lax.axis_index("c")

        def pass_body(f, carry):
            d = wid + f * NW
            pltpu.sync_copy(feat_hbm.at[d], trow)
            pltpu.sync_copy(posb_hbm.at[d], pbv)
            pltpu.async_copy(ids_hbm.at[0], ibufs[0], isems[0])
            pltpu.async_copy(ids_hbm.at[1], ibufs[1], isems[1])

            def do_row(l, ji, js):
                @pl.when(l >= 2)
                def _():
                    pltpu.make_async_copy(
                        stgs[js], out_hbm.at[l - 2, d], wsems[js]).wait()

                pltpu.make_async_copy(
                    ids_hbm.at[0], ibufs[ji], isems[ji]).wait()

                @pl.when(l + 2 < L)
                def _():
                    pltpu.async_copy(
                        ids_hbm.at[l + 2], ibufs[(ji + 2) % NIB],
                        isems[(ji + 2) % NIB])

                pv = pbv[l, pl.ds(0, 16)]

                @plsc.parallel_loop(0, VR, 1, unroll=8)
                def _(k):
                    sl = pl.ds(k * 16, 16)
                    iv = ibufs[ji][sl]
                    stgs[js][sl] = plsc.load_gather(trow, [iv]) + pv

                pltpu.async_copy(stgs[js], out_hbm.at[l, d], wsems[js])

            def row_group(go, carry2):
                for j in range(STEP):
                    do_row(go * STEP + j, j % NIB, j % NST)
                return carry2

            lax.fori_loop(0, L // STEP, row_group, 0)
            for l in range(L - L % STEP, L):
                do_row(l, l % NIB, l % NST)
            pltpu.make_async_copy(
                stgs[(L - 2) % NST], out_hbm.at[L - 2, d],
                wsems[(L - 2) % NST]).wait()
            pltpu.make_async_copy(
                stgs[(L - 1) % NST], out_hbm.at[L - 1, d],
                wsems[(L - 1) % NST]).wait()
            return carry

        lax.fori_loop(0, PASSES, pass_body, 0)

    return gather_add


def kernel(phrase, phrase_emb_weight, pos_emb_weight):
    Bsz, Lseq = phrase.shape
    ids = phrase.T.astype(jnp.int32)
    feat = phrase_emb_weight.T
    pos = pos_emb_weight[:Lseq]
    posb = jnp.broadcast_to(pos.T[:, :, None], (pos.shape[1], Lseq, 16))
    out_phys = _build_kernel()(ids, feat, posb)
    return jnp.transpose(out_phys, (2, 0, 1))
